# baseline (speedup 1.0000x reference)
import functools

import jax
import jax.numpy as jnp
from jax import lax
from jax.experimental import pallas as pl
from jax.experimental.pallas import tpu as pltpu
from jax.experimental.pallas import tpu_sc as plsc

_NBUF = 4
_G_AHEAD = 3


def kernel(tokens, wte_weight, learned_embedding):
    B, S = tokens.shape
    V, D = wte_weight.shape
    P = learned_embedding.shape[0]
    DG = D // 8
    tok_t = tokens.astype(jnp.int32).T

    info = plsc.get_sparse_core_info()
    NC, NS = info.num_cores, info.num_subcores
    NW = NC * NS
    BT = B // 128
    assert BT == NW and S % _NBUF == 0 and P == _NBUF + 1

    mesh = plsc.VectorSubcoreMesh(core_axis_name="c", subcore_axis_name="s")

    @functools.partial(
        pl.kernel,
        mesh=mesh,
        out_type=jax.ShapeDtypeStruct((S, DG, BT, 8, 128), jnp.float32),
        scratch_types=(
            [pltpu.VMEM((S, 128), jnp.int32)]
            + [pltpu.VMEM((128, D), jnp.float32) for _ in range(_NBUF)]
            + [pltpu.VMEM((DG, 1, 8, 128), jnp.float32) for _ in range(_NBUF)]
            + [
                pltpu.VMEM((P, D), jnp.float32),
                pltpu.SemaphoreType.DMA((_NBUF,)),
                pltpu.SemaphoreType.DMA((_NBUF,)),
            ]
        ),
        compiler_params=pltpu.CompilerParams(
            use_tc_tiling_on_sc=False, needs_layout_passes=False
        ),
    )
    def run(tok_hbm, wte_hbm, learned_hbm, out_hbm, *scratch):
        idx_v = scratch[0]
        g_v = scratch[1 : 1 + _NBUF]
        t_v = scratch[1 + _NBUF : 1 + 2 * _NBUF]
        le_v = scratch[1 + 2 * _NBUF]
        sem_g, sem_o = scratch[2 + 2 * _NBUF :]
        wid = lax.axis_index("s") * NC + lax.axis_index("c")

        pltpu.sync_copy(learned_hbm, le_v)
        pltpu.sync_copy(tok_hbm.at[:, pl.ds(wid * 128, 128)], idx_v)

        iota16 = lax.iota(jnp.int32, 16)
        row_idx = [16 * g + iota16 for g in range(8)]

        def gather_desc(c, p):
            return pltpu.make_async_copy(
                wte_hbm.at[idx_v.at[c]], g_v[p], sem_g.at[p]
            )

        def write_desc(c, k):
            return pltpu.make_async_copy(
                t_v[k], out_hbm.at[c, pl.ds(0, DG), pl.ds(wid, 1)], sem_o.at[k]
            )

        def transpose_block(p, k):
            def tbody(dg, carry):
                for dr in range(8):
                    d = 8 * dg + dr
                    col_idx = jnp.full((16,), 0, jnp.int32) + d
                    for g in range(8):
                        t_v[k][dg, 0, dr, pl.ds(16 * g, 16)] = plsc.load_gather(
                            g_v[p], [row_idx[g], col_idx]
                        )
                return carry

            lax.fori_loop(0, DG, tbody, 0)

        def prefix_block(c, k):
            def pbody(dg, carry):
                for dr in range(8):
                    d = 8 * dg + dr
                    val = plsc.load_gather(
                        le_v,
                        [
                            jnp.full((16,), c, jnp.int32),
                            jnp.full((16,), 0, jnp.int32) + d,
                        ],
                    )
                    for g in range(8):
                        t_v[k][dg, 0, dr, pl.ds(16 * g, 16)] = val
                return carry

            lax.fori_loop(0, DG, pbody, 0)

        for c in range(_G_AHEAD):
            gather_desc(c, c % _NBUF).start()

        def phase(c, k, wait_prev_write, start_gather, prefix_mode):
            gather_desc(c, k).wait()
            if wait_prev_write:
                write_desc(c - 2, (k - 2) % _NBUF).wait()
            if prefix_mode == "static":
                prefix_block(c, k)
            elif prefix_mode == "maybe":
                pl.when(c == P - 1)(lambda: prefix_block(c, k))
                pl.when(c != P - 1)(lambda: transpose_block(k, k))
            else:
                transpose_block(k, k)
            write_desc(c, k).start()
            if start_gather:
                gather_desc(c + _G_AHEAD, (k + _G_AHEAD) % _NBUF).start()

        for c in range(_NBUF):
            phase(c, c, c >= 2, c + _G_AHEAD < S, "static")

        def body(j, carry):
            c0 = j * _NBUF
            phase(c0, 0, True, True, "maybe")
            for k in range(1, _NBUF):
                phase(c0 + k, k, True, True, "no")
            return carry

        lax.fori_loop(1, S // _NBUF - 1, body, 0)

        c0 = S - _NBUF
        for k in range(_NBUF):
            phase(c0 + k, k, True, c0 + k + _G_AHEAD < S, "no")
        write_desc(S - 2, (_NBUF - 2) % _NBUF).wait()
        write_desc(S - 1, _NBUF - 1).wait()

    out5 = run(tok_t, wte_weight, learned_embedding)
    return out5.transpose(2, 4, 0, 1, 3).reshape(B, S, D)


# speedup vs baseline: 5.8941x; 5.8941x over previous
import functools

import jax
import jax.numpy as jnp
from jax import lax
from jax.experimental import pallas as pl
from jax.experimental.pallas import tpu as pltpu
from jax.experimental.pallas import tpu_sc as plsc

_NBUF = 4
_G_AHEAD = 3


def kernel(tokens, wte_weight, learned_embedding):
    B, S = tokens.shape
    V, D = wte_weight.shape
    P = learned_embedding.shape[0]
    DG = D // 8
    tok_t = tokens.astype(jnp.int32).T

    info = plsc.get_sparse_core_info()
    NC, NS = info.num_cores, info.num_subcores
    NW = NC * NS
    BT = B // 128
    assert BT == NW and S % _NBUF == 0 and P == _NBUF + 1

    mesh = plsc.VectorSubcoreMesh(core_axis_name="c", subcore_axis_name="s")

    @functools.partial(
        pl.kernel,
        mesh=mesh,
        out_type=jax.ShapeDtypeStruct((S, DG, BT, 8, 128), jnp.float32),
        scratch_types=(
            [pltpu.VMEM((S, 128), jnp.int32)]
            + [pltpu.VMEM((128, D), jnp.float32) for _ in range(_NBUF)]
            + [pltpu.VMEM((DG, 1, 8, 128), jnp.float32) for _ in range(_NBUF)]
            + [
                pltpu.VMEM((P, D), jnp.float32),
                pltpu.SemaphoreType.DMA((_NBUF,)),
                pltpu.SemaphoreType.DMA((_NBUF,)),
            ]
        ),
        compiler_params=pltpu.CompilerParams(
            use_tc_tiling_on_sc=False, needs_layout_passes=False
        ),
    )
    def run(tok_hbm, wte_hbm, learned_hbm, out_hbm, *scratch):
        idx_v = scratch[0]
        g_v = scratch[1 : 1 + _NBUF]
        t_v = scratch[1 + _NBUF : 1 + 2 * _NBUF]
        le_v = scratch[1 + 2 * _NBUF]
        sem_g, sem_o = scratch[2 + 2 * _NBUF :]
        wid = lax.axis_index("s") * NC + lax.axis_index("c")

        pltpu.sync_copy(learned_hbm, le_v)
        pltpu.sync_copy(tok_hbm.at[:, pl.ds(wid * 128, 128)], idx_v)

        iota16 = lax.iota(jnp.int32, 16)
        row_idx = [16 * g + iota16 for g in range(8)]

        def gather_desc(c, p):
            return pltpu.make_async_copy(
                wte_hbm.at[idx_v.at[c]], g_v[p], sem_g.at[p]
            )

        def write_desc(c, k):
            return pltpu.make_async_copy(
                t_v[k], out_hbm.at[c, pl.ds(0, DG), pl.ds(wid, 1)], sem_o.at[k]
            )

        def transpose_block(p, k):
            @functools.partial(plsc.parallel_loop, 0, DG)
            def _(dg):
                for dr in range(8):
                    d = 8 * dg + dr
                    col_idx = jnp.full((16,), 0, jnp.int32) + d
                    for g in range(8):
                        t_v[k][dg, 0, dr, pl.ds(16 * g, 16)] = plsc.load_gather(
                            g_v[p], [row_idx[g], col_idx]
                        )

        def prefix_block(c, k):
            def pbody(dg, carry):
                for dr in range(8):
                    d = 8 * dg + dr
                    val = plsc.load_gather(
                        le_v,
                        [
                            jnp.full((16,), c, jnp.int32),
                            jnp.full((16,), 0, jnp.int32) + d,
                        ],
                    )
                    for g in range(8):
                        t_v[k][dg, 0, dr, pl.ds(16 * g, 16)] = val
                return carry

            lax.fori_loop(0, DG, pbody, 0)

        for c in range(_G_AHEAD):
            gather_desc(c, c % _NBUF).start()

        def phase(c, k, wait_prev_write, start_gather, prefix_mode):
            gather_desc(c, k).wait()
            if wait_prev_write:
                write_desc(c - 2, (k - 2) % _NBUF).wait()
            if prefix_mode == "static":
                prefix_block(c, k)
            elif prefix_mode == "maybe":
                pl.when(c == P - 1)(lambda: prefix_block(c, k))
                pl.when(c != P - 1)(lambda: transpose_block(k, k))
            else:
                transpose_block(k, k)
            write_desc(c, k).start()
            if start_gather:
                gather_desc(c + _G_AHEAD, (k + _G_AHEAD) % _NBUF).start()

        for c in range(_NBUF):
            phase(c, c, c >= 2, c + _G_AHEAD < S, "static")

        def body(j, carry):
            c0 = j * _NBUF
            phase(c0, 0, True, True, "maybe")
            for k in range(1, _NBUF):
                phase(c0 + k, k, True, True, "no")
            return carry

        lax.fori_loop(1, S // _NBUF - 1, body, 0)

        c0 = S - _NBUF
        for k in range(_NBUF):
            phase(c0 + k, k, True, c0 + k + _G_AHEAD < S, "no")
        write_desc(S - 2, (_NBUF - 2) % _NBUF).wait()
        write_desc(S - 1, _NBUF - 1).wait()

    out5 = run(tok_t, wte_weight, learned_embedding)
    return out5.transpose(2, 4, 0, 1, 3).reshape(B, S, D)
